# baseline (speedup 1.0000x reference)
import functools

import jax
import jax.numpy as jnp
from jax import lax
from jax.experimental import pallas as pl
from jax.experimental.pallas import tpu as pltpu
from jax.experimental.pallas import tpu_sc as plsc

N, E, D, H, O = 10000, 320000, 128, 64, 64
NP = 10240
NC, NS = 2, 16
BATCH = 128
NB0, NB1 = 32, 128
NBMAX = max(NB0, NB1)
EP = NS * (NB0 + NB1) * BATCH
ROWS_PT = NP // NS


CW = 8
NBUF = 4


def _sc_body(y_hbm, src_hbm, dst_hbm, zrows_hbm, zcnt_hbm, ones_hbm,
             sums_hbm, cnt_hbm,
             src_v, dst_v, buf0, buf1, buf2, buf3, ones_v, acc, cacc,
             gsem0, gsem1, gsem2, gsem3, ssem0, ssem1, ssem2, ssem3, csem):
  cid = lax.axis_index("c")
  sid = lax.axis_index("s")
  nb = lax.select(cid == 0, jnp.int32(NB0), jnp.int32(NB1))
  bufs = [buf0, buf1, buf2, buf3]
  gsems = [gsem0, gsem1, gsem2, gsem3]
  ssems = [ssem0, ssem1, ssem2, ssem3]

  pltpu.sync_copy(zrows_hbm, acc.at[pl.ds(sid * ROWS_PT, ROWS_PT)])
  pltpu.sync_copy(zcnt_hbm, cacc.at[pl.ds(sid * ROWS_PT, ROWS_PT)])
  pltpu.sync_copy(ones_hbm, ones_v)
  pltpu.sync_copy(src_hbm.at[cid, sid], src_v)
  pltpu.sync_copy(dst_hbm.at[cid, sid], dst_v)
  plsc.subcore_barrier()

  for k in range(NBUF):
    pltpu.async_copy(y_hbm.at[src_v.at[k]], bufs[k], gsems[k])

  def _quad(i, carry):
    for k in range(NBUF):
      b = NBUF * i + k
      pltpu.make_async_copy(y_hbm.at[src_v.at[b]], bufs[k], gsems[k]).wait()
      pltpu.async_copy(bufs[k], acc.at[dst_v.at[b]], ssems[k], add=True)
      pltpu.async_copy(ones_v, cacc.at[dst_v.at[b]], csem, add=True)

      @pl.when(i < nb // NBUF - 1)
      def _():
        pltpu.make_async_copy(bufs[k], acc.at[dst_v.at[b]], ssems[k]).wait()
        pltpu.async_copy(y_hbm.at[src_v.at[b + NBUF]], bufs[k], gsems[k])

      @pl.when(i > 0)
      def _():
        pltpu.make_async_copy(ones_v, cacc.at[dst_v.at[b]], csem).wait()
    return carry

  lax.fori_loop(0, nb // NBUF, _quad, 0)
  for k in range(NBUF):
    pltpu.make_async_copy(bufs[k], acc.at[dst_v.at[nb - NBUF + k]],
                          ssems[k]).wait()
    pltpu.make_async_copy(ones_v, cacc.at[dst_v.at[nb - NBUF + k]],
                          csem).wait()
  plsc.subcore_barrier()

  pltpu.sync_copy(acc.at[pl.ds(sid * ROWS_PT, ROWS_PT)],
                  sums_hbm.at[cid, pl.ds(sid * ROWS_PT, ROWS_PT)])
  pltpu.sync_copy(cacc.at[pl.ds(sid * ROWS_PT, ROWS_PT)],
                  cnt_hbm.at[cid, pl.ds(sid * ROWS_PT, ROWS_PT)])


_sc_segment_mean_parts = functools.partial(
    pl.kernel,
    out_type=[
        jax.ShapeDtypeStruct((NC, NP, H), jnp.float32),
        jax.ShapeDtypeStruct((NC, NP, CW), jnp.float32),
    ],
    mesh=plsc.VectorSubcoreMesh(core_axis_name="c", subcore_axis_name="s"),
    compiler_params=pltpu.CompilerParams(use_tc_tiling_on_sc=False),
    scratch_types=[
        pltpu.VMEM((NBMAX, BATCH), jnp.int32),
        pltpu.VMEM((NBMAX, BATCH), jnp.int32),
        pltpu.VMEM((BATCH, H), jnp.float32),
        pltpu.VMEM((BATCH, H), jnp.float32),
        pltpu.VMEM((BATCH, H), jnp.float32),
        pltpu.VMEM((BATCH, H), jnp.float32),
        pltpu.VMEM((BATCH, CW), jnp.float32),
        pltpu.VMEM_SHARED((NP, H), jnp.float32),
        pltpu.VMEM_SHARED((NP, CW), jnp.float32),
    ] + [pltpu.SemaphoreType.DMA] * 9,
)(_sc_body)


def _mm_body(x_ref, wl_ref, wr_ref, y_ref, z_ref):
  xb = x_ref[...]
  y_ref[...] = jnp.dot(xb, wl_ref[...], preferred_element_type=jnp.float32)
  z_ref[...] = jnp.dot(xb, wr_ref[...], preferred_element_type=jnp.float32)


def _tc_in_proj(x, W_l, W_r):
  blk = N // 10
  return pl.pallas_call(
      _mm_body,
      grid=(10,),
      in_specs=[
          pl.BlockSpec((blk, D), lambda i: (i, 0)),
          pl.BlockSpec((D, H), lambda i: (0, 0)),
          pl.BlockSpec((D, H), lambda i: (0, 0)),
      ],
      out_specs=[
          pl.BlockSpec((blk, H), lambda i: (i, 0)),
          pl.BlockSpec((blk, H), lambda i: (i, 0)),
      ],
      out_shape=[
          jax.ShapeDtypeStruct((N, H), jnp.float32),
          jax.ShapeDtypeStruct((N, H), jnp.float32),
      ],
      compiler_params=pltpu.CompilerParams(
          dimension_semantics=("parallel",)),
  )(x, W_l, W_r)


def _out_body(sums_ref, cnt_ref, z_ref, bl_ref, wlin_ref, blin_ref, o_ref):
  s = sums_ref[0] + sums_ref[1]
  c = (cnt_ref[0] + cnt_ref[1])[:, 0:1]
  mean = s / jnp.maximum(c, 1.0)
  h = mean + bl_ref[...] + z_ref[...]
  h = jnp.where(h > 0.0, h, jnp.exp(jnp.minimum(h, 0.0)) - 1.0)
  o_ref[...] = (jnp.dot(h, wlin_ref[...], preferred_element_type=jnp.float32)
                + blin_ref[...])


def _tc_out_proj(sums, cnts, z, b_l, W_lin, b_lin):
  blk = N // 10
  return pl.pallas_call(
      _out_body,
      grid=(10,),
      in_specs=[
          pl.BlockSpec((NC, blk, H), lambda i: (0, i, 0)),
          pl.BlockSpec((NC, blk, CW), lambda i: (0, i, 0)),
          pl.BlockSpec((blk, H), lambda i: (i, 0)),
          pl.BlockSpec((1, H), lambda i: (0, 0)),
          pl.BlockSpec((H, O), lambda i: (0, 0)),
          pl.BlockSpec((1, O), lambda i: (0, 0)),
      ],
      out_specs=pl.BlockSpec((blk, O), lambda i: (i, 0)),
      out_shape=jax.ShapeDtypeStruct((N, O), jnp.float32),
      compiler_params=pltpu.CompilerParams(
          dimension_semantics=("parallel",)),
  )(sums, cnts, z, b_l.reshape(1, H), W_lin, b_lin.reshape(1, O))


def kernel(x, edge_index, W_l, b_l, W_r, W_lin, b_lin):
  y, z = _tc_in_proj(x, W_l, W_r)

  pad_e = EP - E
  e0 = NS * NB0 * BATCH
  src_f = jnp.concatenate([edge_index[0], jnp.zeros((pad_e,), jnp.int32)])
  dst_f = jnp.concatenate([edge_index[1], jnp.full((pad_e,), N, jnp.int32)])
  src_p = jnp.zeros((NC, NS, NBMAX, BATCH), jnp.int32)
  src_p = src_p.at[0, :, :NB0].set(src_f[:e0].reshape(NS, NB0, BATCH))
  src_p = src_p.at[1, :, :NB1].set(src_f[e0:].reshape(NS, NB1, BATCH))
  dst_p = jnp.full((NC, NS, NBMAX, BATCH), N, jnp.int32)
  dst_p = dst_p.at[0, :, :NB0].set(dst_f[:e0].reshape(NS, NB0, BATCH))
  dst_p = dst_p.at[1, :, :NB1].set(dst_f[e0:].reshape(NS, NB1, BATCH))

  zrows = jnp.zeros((ROWS_PT, H), jnp.float32)
  zcnt = jnp.zeros((ROWS_PT, CW), jnp.float32)
  ones_rows = jnp.zeros((BATCH, CW), jnp.float32).at[:, 0].set(1.0)
  sums, cnts = _sc_segment_mean_parts(y, src_p, dst_p, zrows, zcnt, ones_rows)

  return _tc_out_proj(sums, cnts, z, b_l, W_lin, b_lin)


# speedup vs baseline: 1.1104x; 1.1104x over previous
import functools

import jax
import jax.numpy as jnp
from jax import lax
from jax.experimental import pallas as pl
from jax.experimental.pallas import tpu as pltpu
from jax.experimental.pallas import tpu_sc as plsc

N, E, D, H, O = 10000, 320000, 128, 64, 64
NP = 10240
NC, NS = 2, 16
BATCH = 128
NB0, NB1 = 128, 32
NBMAX = max(NB0, NB1)
EP = NS * (NB0 + NB1) * BATCH
ROWS_PT = NP // NS


CW = 8
NBUF = 4


def _sc_body(y_hbm, src_hbm, dst_hbm, zrows_hbm, zcnt_hbm, ones_hbm,
             sums_hbm, cnt_hbm,
             src_v, dst_v, buf0, buf1, buf2, buf3, ones_v, acc, cacc,
             gsem0, gsem1, gsem2, gsem3, ssem0, ssem1, ssem2, ssem3, csem):
  cid = lax.axis_index("c")
  sid = lax.axis_index("s")
  nb = lax.select(cid == 0, jnp.int32(NB0), jnp.int32(NB1))
  bufs = [buf0, buf1, buf2, buf3]
  gsems = [gsem0, gsem1, gsem2, gsem3]
  ssems = [ssem0, ssem1, ssem2, ssem3]

  pltpu.sync_copy(zrows_hbm, acc.at[pl.ds(sid * ROWS_PT, ROWS_PT)])
  pltpu.sync_copy(zcnt_hbm, cacc.at[pl.ds(sid * ROWS_PT, ROWS_PT)])
  pltpu.sync_copy(ones_hbm, ones_v)
  pltpu.sync_copy(src_hbm.at[cid, sid], src_v)
  pltpu.sync_copy(dst_hbm.at[cid, sid], dst_v)
  plsc.subcore_barrier()

  for k in range(NBUF):
    pltpu.async_copy(y_hbm.at[src_v.at[k]], bufs[k], gsems[k])

  def _quad(i, carry):
    for k in range(NBUF):
      b = NBUF * i + k
      pltpu.make_async_copy(y_hbm.at[src_v.at[b]], bufs[k], gsems[k]).wait()
      pltpu.async_copy(bufs[k], acc.at[dst_v.at[b]], ssems[k], add=True)
      pltpu.async_copy(ones_v, cacc.at[dst_v.at[b]], csem, add=True)

      @pl.when(i < nb // NBUF - 1)
      def _():
        pltpu.make_async_copy(bufs[k], acc.at[dst_v.at[b]], ssems[k]).wait()
        pltpu.async_copy(y_hbm.at[src_v.at[b + NBUF]], bufs[k], gsems[k])

      @pl.when(i > 0)
      def _():
        pltpu.make_async_copy(ones_v, cacc.at[dst_v.at[b]], csem).wait()
    return carry

  lax.fori_loop(0, nb // NBUF, _quad, 0)
  for k in range(NBUF):
    pltpu.make_async_copy(bufs[k], acc.at[dst_v.at[nb - NBUF + k]],
                          ssems[k]).wait()
    pltpu.make_async_copy(ones_v, cacc.at[dst_v.at[nb - NBUF + k]],
                          csem).wait()
  plsc.subcore_barrier()

  pltpu.sync_copy(acc.at[pl.ds(sid * ROWS_PT, ROWS_PT)],
                  sums_hbm.at[cid, pl.ds(sid * ROWS_PT, ROWS_PT)])
  pltpu.sync_copy(cacc.at[pl.ds(sid * ROWS_PT, ROWS_PT)],
                  cnt_hbm.at[cid, pl.ds(sid * ROWS_PT, ROWS_PT)])


_sc_segment_mean_parts = functools.partial(
    pl.kernel,
    out_type=[
        jax.ShapeDtypeStruct((NC, NP, H), jnp.float32),
        jax.ShapeDtypeStruct((NC, NP, CW), jnp.float32),
    ],
    mesh=plsc.VectorSubcoreMesh(core_axis_name="c", subcore_axis_name="s"),
    compiler_params=pltpu.CompilerParams(use_tc_tiling_on_sc=False),
    scratch_types=[
        pltpu.VMEM((NBMAX, BATCH), jnp.int32),
        pltpu.VMEM((NBMAX, BATCH), jnp.int32),
        pltpu.VMEM((BATCH, H), jnp.float32),
        pltpu.VMEM((BATCH, H), jnp.float32),
        pltpu.VMEM((BATCH, H), jnp.float32),
        pltpu.VMEM((BATCH, H), jnp.float32),
        pltpu.VMEM((BATCH, CW), jnp.float32),
        pltpu.VMEM_SHARED((NP, H), jnp.float32),
        pltpu.VMEM_SHARED((NP, CW), jnp.float32),
    ] + [pltpu.SemaphoreType.DMA] * 9,
)(_sc_body)


def _mm_body(x_ref, wl_ref, wr_ref, y_ref, z_ref):
  xb = x_ref[...]
  y_ref[...] = jnp.dot(xb, wl_ref[...], preferred_element_type=jnp.float32)
  z_ref[...] = jnp.dot(xb, wr_ref[...], preferred_element_type=jnp.float32)


def _tc_in_proj(x, W_l, W_r):
  blk = N // 10
  return pl.pallas_call(
      _mm_body,
      grid=(10,),
      in_specs=[
          pl.BlockSpec((blk, D), lambda i: (i, 0)),
          pl.BlockSpec((D, H), lambda i: (0, 0)),
          pl.BlockSpec((D, H), lambda i: (0, 0)),
      ],
      out_specs=[
          pl.BlockSpec((blk, H), lambda i: (i, 0)),
          pl.BlockSpec((blk, H), lambda i: (i, 0)),
      ],
      out_shape=[
          jax.ShapeDtypeStruct((N, H), jnp.float32),
          jax.ShapeDtypeStruct((N, H), jnp.float32),
      ],
      compiler_params=pltpu.CompilerParams(
          dimension_semantics=("parallel",)),
  )(x, W_l, W_r)


def _out_body(sums_ref, cnt_ref, z_ref, bl_ref, wlin_ref, blin_ref, o_ref):
  s = sums_ref[0] + sums_ref[1]
  c = (cnt_ref[0] + cnt_ref[1])[:, 0:1]
  mean = s / jnp.maximum(c, 1.0)
  h = mean + bl_ref[...] + z_ref[...]
  h = jnp.where(h > 0.0, h, jnp.exp(jnp.minimum(h, 0.0)) - 1.0)
  o_ref[...] = (jnp.dot(h, wlin_ref[...], preferred_element_type=jnp.float32)
                + blin_ref[...])


def _tc_out_proj(sums, cnts, z, b_l, W_lin, b_lin):
  blk = N // 10
  return pl.pallas_call(
      _out_body,
      grid=(10,),
      in_specs=[
          pl.BlockSpec((NC, blk, H), lambda i: (0, i, 0)),
          pl.BlockSpec((NC, blk, CW), lambda i: (0, i, 0)),
          pl.BlockSpec((blk, H), lambda i: (i, 0)),
          pl.BlockSpec((1, H), lambda i: (0, 0)),
          pl.BlockSpec((H, O), lambda i: (0, 0)),
          pl.BlockSpec((1, O), lambda i: (0, 0)),
      ],
      out_specs=pl.BlockSpec((blk, O), lambda i: (i, 0)),
      out_shape=jax.ShapeDtypeStruct((N, O), jnp.float32),
      compiler_params=pltpu.CompilerParams(
          dimension_semantics=("parallel",)),
  )(sums, cnts, z, b_l.reshape(1, H), W_lin, b_lin.reshape(1, O))


def kernel(x, edge_index, W_l, b_l, W_r, W_lin, b_lin):
  y, z = _tc_in_proj(x, W_l, W_r)

  pad_e = EP - E
  e0 = NS * NB0 * BATCH
  src_f = jnp.concatenate([edge_index[0], jnp.zeros((pad_e,), jnp.int32)])
  dst_f = jnp.concatenate([edge_index[1], jnp.full((pad_e,), N, jnp.int32)])
  src_p = jnp.zeros((NC, NS, NBMAX, BATCH), jnp.int32)
  src_p = src_p.at[0, :, :NB0].set(src_f[:e0].reshape(NS, NB0, BATCH))
  src_p = src_p.at[1, :, :NB1].set(src_f[e0:].reshape(NS, NB1, BATCH))
  dst_p = jnp.full((NC, NS, NBMAX, BATCH), N, jnp.int32)
  dst_p = dst_p.at[0, :, :NB0].set(dst_f[:e0].reshape(NS, NB0, BATCH))
  dst_p = dst_p.at[1, :, :NB1].set(dst_f[e0:].reshape(NS, NB1, BATCH))

  zrows = jnp.zeros((ROWS_PT, H), jnp.float32)
  zcnt = jnp.zeros((ROWS_PT, CW), jnp.float32)
  ones_rows = jnp.zeros((BATCH, CW), jnp.float32).at[:, 0].set(1.0)
  sums, cnts = _sc_segment_mean_parts(y, src_p, dst_p, zrows, zcnt, ones_rows)

  return _tc_out_proj(sums, cnts, z, b_l, W_lin, b_lin)
